# baseline (speedup 1.0000x reference)
import functools

import jax
import jax.numpy as jnp
from jax import lax
from jax.experimental import pallas as pl
from jax.experimental.pallas import tpu as pltpu
from jax.experimental.pallas import tpu_sc as plsc

N = 10000
E = 320000
D = 128
EPS = 1e-5

NC = 2
NS = 16
NW = NC * NS

EPW = E // NW
KE = 125
CH = 40
NCHUNK = EPW // (KE * CH)
KD = 125
ITERS_D = EPW // KD
NP = 10240
DSLICE = NP // NS
NT10 = 10
RPT = N // NT10
ZR = 200

_mesh = plsc.VectorSubcoreMesh(core_axis_name="c", subcore_axis_name="s")


@functools.partial(
    pl.kernel,
    out_type=jax.ShapeDtypeStruct((2 * NP,), jnp.float32),
    mesh=_mesh,
    scratch_types=[
        pltpu.VMEM((ITERS_D, KD), jnp.int32),
        pltpu.VMEM((KD,), jnp.float32),
        pltpu.VMEM_SHARED((NP,), jnp.float32),
        pltpu.SemaphoreType.DMA,
    ],
)
def _deg_sc(dst3_hbm, ones_hbm, zeros_hbm, out_hbm, idx_v, ones_v, deg_sh,
            sem):
    c = lax.axis_index("c")
    s = lax.axis_index("s")
    wid = s * NC + c
    pltpu.sync_copy(dst3_hbm.at[wid], idx_v)
    pltpu.sync_copy(ones_hbm, ones_v)
    pltpu.sync_copy(zeros_hbm, deg_sh.at[pl.ds(s * DSLICE, DSLICE)])
    plsc.subcore_barrier()

    def body(j, carry):
        for b in range(8):
            pltpu.async_copy(ones_v, deg_sh.at[idx_v.at[8 * j + b]], sem,
                             add=True)
        for b in range(8):
            pltpu.make_async_copy(ones_v, deg_sh.at[idx_v.at[0]],
                                  sem).wait()
        return carry

    lax.fori_loop(0, ITERS_D // 8, body, 0)
    plsc.subcore_barrier()
    pltpu.sync_copy(deg_sh.at[pl.ds(s * DSLICE, DSLICE)],
                    out_hbm.at[pl.ds(c * NP + s * DSLICE, DSLICE)])


@functools.partial(
    pl.kernel,
    out_type=jax.ShapeDtypeStruct((2 * N, D), jnp.float32),
    mesh=_mesh,
    scratch_types=[
        pltpu.VMEM((CH, KE), jnp.int32),
        pltpu.VMEM((CH, KE), jnp.int32),
        [pltpu.VMEM((KE, D), jnp.float32)] * 2,
        pltpu.VMEM_SHARED((N, D), jnp.float32),
        [pltpu.SemaphoreType.DMA] * 2,
    ],
)
def _scatter_sc(g_hbm, src4_hbm, dst4_hbm, zrows_hbm, out_hbm,
                idx_s, idx_d, rows, acc_sh, semg):
    c = lax.axis_index("c")
    s = lax.axis_index("s")
    wid = s * NC + c

    @pl.when(s < NT10)
    def _():
        for j in range(RPT // ZR):
            pltpu.async_copy(zrows_hbm,
                             acc_sh.at[pl.ds(s * RPT + j * ZR, ZR)],
                             semg[j % 2])
        for j in range(RPT // ZR):
            pltpu.make_async_copy(zrows_hbm, acc_sh.at[pl.ds(0, ZR)],
                                  semg[j % 2]).wait()

    plsc.subcore_barrier()

    for ck in range(NCHUNK):
        pltpu.sync_copy(src4_hbm.at[wid, ck], idx_s)
        pltpu.sync_copy(dst4_hbm.at[wid, ck], idx_d)
        for b in range(2):
            pltpu.async_copy(g_hbm.at[idx_s.at[b]], rows[b], semg[b])

        def body(j, carry):
            for b in range(2):
                i = 2 * j + b
                pltpu.make_async_copy(g_hbm.at[idx_s.at[0]], rows[b],
                                      semg[b]).wait()
                pltpu.sync_copy(rows[b], acc_sh.at[idx_d.at[i]],
                                add=True)
                pltpu.async_copy(g_hbm.at[idx_s.at[i + 2]], rows[b],
                                 semg[b])
            return carry

        lax.fori_loop(0, CH // 2 - 1, body, 0)
        for b in range(2):
            i = CH - 2 + b
            pltpu.make_async_copy(g_hbm.at[idx_s.at[0]], rows[b],
                                  semg[b]).wait()
            pltpu.sync_copy(rows[b], acc_sh.at[idx_d.at[i]], add=True)

    plsc.subcore_barrier()

    @pl.when(s < NT10)
    def _():
        for j in range(RPT // ZR):
            r0 = s * RPT + j * ZR
            pltpu.async_copy(acc_sh.at[pl.ds(r0, ZR)],
                             out_hbm.at[pl.ds(c * N + r0, ZR)],
                             semg[j % 2])
        for j in range(RPT // ZR):
            pltpu.make_async_copy(acc_sh.at[pl.ds(0, ZR)],
                                  out_hbm.at[pl.ds(0, ZR)],
                                  semg[j % 2]).wait()


_BR = 1000
_GRID = N // _BR


def _k1_body(deg_ref, x_ref, w1_ref, g1_ref, dinv_ref):
    deg = deg_ref[:, 0] + deg_ref[:, 1] + 1.0
    dinv = lax.rsqrt(jnp.maximum(deg, 1.0))
    dinv_ref[...] = dinv[:, None]
    h = jnp.dot(x_ref[...], w1_ref[...], preferred_element_type=jnp.float32)
    g1_ref[...] = h * dinv[:, None]


def _k2_body(pa_ref, pb_ref, g_ref, dinv_ref, b_ref, w_ref, h_ref, gn_ref):
    dinv = dinv_ref[...]
    h = jax.nn.relu(dinv * (pa_ref[...] + pb_ref[...] + g_ref[...])
                    + b_ref[...])
    h_ref[...] = h
    gn_ref[...] = jnp.dot(h, w_ref[...],
                          preferred_element_type=jnp.float32) * dinv


def _k3_body(pa_ref, pb_ref, g_ref, dinv_ref, b_ref, x_ref, h1_ref,
             h2_ref, stats_ref):
    i = pl.program_id(0)
    dinv = dinv_ref[...]
    h2 = jax.nn.relu(dinv * (pa_ref[...] + pb_ref[...] + g_ref[...])
                     + b_ref[...])
    h2_ref[...] = h2
    x = x_ref[...]
    h1 = h1_ref[...]
    sums = jnp.concatenate([x.sum(0), h1.sum(0), h2.sum(0)])[None, :]
    sqs = jnp.concatenate([(x * x).sum(0), (h1 * h1).sum(0),
                           (h2 * h2).sum(0)])[None, :]
    new = jnp.concatenate([sums, sqs], axis=0)

    @pl.when(i == 0)
    def _():
        stats_ref[...] = new

    @pl.when(i > 0)
    def _():
        stats_ref[...] += new


def _k5_body(x_ref, h1_ref, h2_ref, wo_ref, stats_ref, gamma_ref,
             beta_ref, bo_ref, y_ref):
    inv_n = 1.0 / N
    mean = stats_ref[0:1, :] * inv_n
    ex2 = stats_ref[1:2, :] * inv_n
    var = ex2 - mean * mean
    wo = wo_ref[...]
    sc = gamma_ref[...] * lax.rsqrt(var + EPS)
    beff = bo_ref[...] + jnp.dot(beta_ref[...] - mean * sc, wo,
                                 preferred_element_type=jnp.float32)
    y = jnp.dot(x_ref[...] * sc[:, 0:D], wo[0:D, :],
                preferred_element_type=jnp.float32)
    y += jnp.dot(h1_ref[...] * sc[:, D:2 * D], wo[D:2 * D, :],
                 preferred_element_type=jnp.float32)
    y += jnp.dot(h2_ref[...] * sc[:, 2 * D:3 * D], wo[2 * D:3 * D, :],
                 preferred_element_type=jnp.float32)
    y_ref[...] = y + beff


def _rows_spec(i):
    return (i, 0)


def _all_spec(i):
    return (0, 0)


_f32 = jnp.float32


def _k1(deg2, x, w1):
    return pl.pallas_call(
        _k1_body,
        grid=(_GRID,),
        in_specs=[
            pl.BlockSpec((_BR, 2), _rows_spec),
            pl.BlockSpec((_BR, D), _rows_spec),
            pl.BlockSpec((D, D), _all_spec),
        ],
        out_specs=[
            pl.BlockSpec((_BR, D), _rows_spec),
            pl.BlockSpec((_BR, 1), _rows_spec),
        ],
        out_shape=[
            jax.ShapeDtypeStruct((N, D), _f32),
            jax.ShapeDtypeStruct((N, 1), _f32),
        ],
    )(deg2, x, w1)


def _k2(pa, pb, g, dinv, b, w):
    return pl.pallas_call(
        _k2_body,
        grid=(_GRID,),
        in_specs=[
            pl.BlockSpec((_BR, D), _rows_spec),
            pl.BlockSpec((_BR, D), _rows_spec),
            pl.BlockSpec((_BR, D), _rows_spec),
            pl.BlockSpec((_BR, 1), _rows_spec),
            pl.BlockSpec((1, D), _all_spec),
            pl.BlockSpec((D, D), _all_spec),
        ],
        out_specs=[
            pl.BlockSpec((_BR, D), _rows_spec),
            pl.BlockSpec((_BR, D), _rows_spec),
        ],
        out_shape=[
            jax.ShapeDtypeStruct((N, D), _f32),
            jax.ShapeDtypeStruct((N, D), _f32),
        ],
    )(pa, pb, g, dinv, b, w)


def _k3(pa, pb, g, dinv, b, x, h1):
    return pl.pallas_call(
        _k3_body,
        grid=(_GRID,),
        in_specs=[
            pl.BlockSpec((_BR, D), _rows_spec),
            pl.BlockSpec((_BR, D), _rows_spec),
            pl.BlockSpec((_BR, D), _rows_spec),
            pl.BlockSpec((_BR, 1), _rows_spec),
            pl.BlockSpec((1, D), _all_spec),
            pl.BlockSpec((_BR, D), _rows_spec),
            pl.BlockSpec((_BR, D), _rows_spec),
        ],
        out_specs=[
            pl.BlockSpec((_BR, D), _rows_spec),
            pl.BlockSpec((2, 3 * D), _all_spec),
        ],
        out_shape=[
            jax.ShapeDtypeStruct((N, D), _f32),
            jax.ShapeDtypeStruct((2, 3 * D), _f32),
        ],
    )(pa, pb, g, dinv, b, x, h1)


def _k5(x, h1, h2, wo, stats, gamma, beta, bo):
    return pl.pallas_call(
        _k5_body,
        grid=(_GRID,),
        in_specs=[
            pl.BlockSpec((_BR, D), _rows_spec),
            pl.BlockSpec((_BR, D), _rows_spec),
            pl.BlockSpec((_BR, D), _rows_spec),
            pl.BlockSpec((3 * D, D), _all_spec),
            pl.BlockSpec((2, 3 * D), _all_spec),
            pl.BlockSpec((1, 3 * D), _all_spec),
            pl.BlockSpec((1, 3 * D), _all_spec),
            pl.BlockSpec((1, D), _all_spec),
        ],
        out_specs=pl.BlockSpec((_BR, D), _rows_spec),
        out_shape=jax.ShapeDtypeStruct((N, D), _f32),
    )(x, h1, h2, wo, stats, gamma, beta, bo)


def kernel(x, edge_index, W1, b1, W2, b2, bn_gamma, bn_beta, Wo, bo):
    src = edge_index[0]
    dst = edge_index[1]
    src4 = src.reshape(NW, NCHUNK, CH, KE)
    dst4 = dst.reshape(NW, NCHUNK, CH, KE)
    dst3d = dst.reshape(NW, ITERS_D, KD)
    ones_kd = jnp.ones((KD,), _f32)
    zeros_ds = jnp.zeros((DSLICE,), _f32)
    zrows = jnp.zeros((ZR, D), _f32)

    degflat = _deg_sc(dst3d, ones_kd, zeros_ds)
    deg2 = degflat.reshape(2, NP)[:, :N].T

    g1, dinv = _k1(deg2, x, W1)
    p1 = _scatter_sc(g1, src4, dst4, zrows)
    h1, g2 = _k2(p1[:N], p1[N:], g1, dinv, b1.reshape(1, D), W2)
    p2 = _scatter_sc(g2, src4, dst4, zrows)
    h2, stats = _k3(p2[:N], p2[N:], g2, dinv, b2.reshape(1, D),
                    x, h1)
    return _k5(x, h1, h2, Wo, stats, bn_gamma.reshape(1, 3 * D),
               bn_beta.reshape(1, 3 * D), bo.reshape(1, D))


# speedup vs baseline: 1.0107x; 1.0107x over previous
import functools

import jax
import jax.numpy as jnp
from jax import lax
from jax.experimental import pallas as pl
from jax.experimental.pallas import tpu as pltpu
from jax.experimental.pallas import tpu_sc as plsc

N = 10000
E = 320000
D = 128
EPS = 1e-5

NC = 2
NS = 16
NW = NC * NS

EPW = E // NW
KE = 125
CH = 40
NCHUNK = EPW // (KE * CH)
KD = 125
ITERS_D = EPW // KD
NP = 10240
DSLICE = NP // NS
NT10 = 10
RPT = N // NT10
ZR = 200

_mesh = plsc.VectorSubcoreMesh(core_axis_name="c", subcore_axis_name="s")


@functools.partial(
    pl.kernel,
    out_type=jax.ShapeDtypeStruct((2 * NP,), jnp.float32),
    mesh=_mesh,
    scratch_types=[
        pltpu.VMEM((ITERS_D, KD), jnp.int32),
        pltpu.VMEM((KD,), jnp.float32),
        pltpu.VMEM_SHARED((NP,), jnp.float32),
        pltpu.SemaphoreType.DMA,
    ],
)
def _deg_sc(dst3_hbm, ones_hbm, zeros_hbm, out_hbm, idx_v, ones_v, deg_sh,
            sem):
    c = lax.axis_index("c")
    s = lax.axis_index("s")
    wid = s * NC + c
    pltpu.sync_copy(dst3_hbm.at[wid], idx_v)
    pltpu.sync_copy(ones_hbm, ones_v)
    pltpu.sync_copy(zeros_hbm, deg_sh.at[pl.ds(s * DSLICE, DSLICE)])
    plsc.subcore_barrier()

    def body(j, carry):
        for b in range(4):
            pltpu.async_copy(ones_v, deg_sh.at[idx_v.at[4 * j + b]], sem,
                             add=True)
        for b in range(4):
            pltpu.make_async_copy(ones_v, deg_sh.at[idx_v.at[0]],
                                  sem).wait()
        return carry

    lax.fori_loop(0, ITERS_D // 4, body, 0)
    plsc.subcore_barrier()
    pltpu.sync_copy(deg_sh.at[pl.ds(s * DSLICE, DSLICE)],
                    out_hbm.at[pl.ds(c * NP + s * DSLICE, DSLICE)])


@functools.partial(
    pl.kernel,
    out_type=jax.ShapeDtypeStruct((2 * N, D), jnp.float32),
    mesh=_mesh,
    scratch_types=[
        pltpu.VMEM((CH, KE), jnp.int32),
        pltpu.VMEM((CH, KE), jnp.int32),
        [pltpu.VMEM((KE, D), jnp.float32)] * 2,
        pltpu.VMEM_SHARED((N, D), jnp.float32),
        [pltpu.SemaphoreType.DMA] * 2,
    ],
)
def _scatter_sc(g_hbm, src4_hbm, dst4_hbm, zrows_hbm, out_hbm,
                idx_s, idx_d, rows, acc_sh, semg):
    c = lax.axis_index("c")
    s = lax.axis_index("s")
    wid = s * NC + c

    pltpu.sync_copy(src4_hbm.at[wid, 0], idx_s)
    pltpu.sync_copy(dst4_hbm.at[wid, 0], idx_d)
    for b in range(2):
        pltpu.async_copy(g_hbm.at[idx_s.at[b]], rows[b], semg[b])

    @pl.when(s < NT10)
    def _():
        for j in range(RPT // ZR):
            pltpu.sync_copy(zrows_hbm,
                            acc_sh.at[pl.ds(s * RPT + j * ZR, ZR)])

    plsc.subcore_barrier()

    for ck in range(NCHUNK):
        if ck > 0:
            pltpu.sync_copy(src4_hbm.at[wid, ck], idx_s)
            pltpu.sync_copy(dst4_hbm.at[wid, ck], idx_d)
            for b in range(2):
                pltpu.async_copy(g_hbm.at[idx_s.at[b]], rows[b], semg[b])

        def body(j, carry):
            for b in range(2):
                i = 2 * j + b
                pltpu.make_async_copy(g_hbm.at[idx_s.at[0]], rows[b],
                                      semg[b]).wait()
                pltpu.sync_copy(rows[b], acc_sh.at[idx_d.at[i]],
                                add=True)
                pltpu.async_copy(g_hbm.at[idx_s.at[i + 2]], rows[b],
                                 semg[b])
            return carry

        lax.fori_loop(0, CH // 2 - 1, body, 0)
        for b in range(2):
            i = CH - 2 + b
            pltpu.make_async_copy(g_hbm.at[idx_s.at[0]], rows[b],
                                  semg[b]).wait()
            pltpu.sync_copy(rows[b], acc_sh.at[idx_d.at[i]], add=True)

    plsc.subcore_barrier()

    @pl.when(s < NT10)
    def _():
        for j in range(RPT // ZR):
            r0 = s * RPT + j * ZR
            pltpu.sync_copy(acc_sh.at[pl.ds(r0, ZR)],
                            out_hbm.at[pl.ds(c * N + r0, ZR)])


_BR = 1000
_GRID = N // _BR


def _k1_body(deg_ref, x_ref, w1_ref, g1_ref, dinv_ref):
    deg = deg_ref[:, 0] + deg_ref[:, 1] + 1.0
    dinv = lax.rsqrt(jnp.maximum(deg, 1.0))
    dinv_ref[...] = dinv[:, None]
    h = jnp.dot(x_ref[...], w1_ref[...], preferred_element_type=jnp.float32)
    g1_ref[...] = h * dinv[:, None]


def _k2_body(pa_ref, pb_ref, g_ref, dinv_ref, b_ref, w_ref, h_ref, gn_ref):
    dinv = dinv_ref[...]
    h = jax.nn.relu(dinv * (pa_ref[...] + pb_ref[...] + g_ref[...])
                    + b_ref[...])
    h_ref[...] = h
    gn_ref[...] = jnp.dot(h, w_ref[...],
                          preferred_element_type=jnp.float32) * dinv


def _k3_body(pa_ref, pb_ref, g_ref, dinv_ref, b_ref, x_ref, h1_ref,
             h2_ref, stats_ref):
    i = pl.program_id(0)
    dinv = dinv_ref[...]
    h2 = jax.nn.relu(dinv * (pa_ref[...] + pb_ref[...] + g_ref[...])
                     + b_ref[...])
    h2_ref[...] = h2
    x = x_ref[...]
    h1 = h1_ref[...]
    sums = jnp.concatenate([x.sum(0), h1.sum(0), h2.sum(0)])[None, :]
    sqs = jnp.concatenate([(x * x).sum(0), (h1 * h1).sum(0),
                           (h2 * h2).sum(0)])[None, :]
    new = jnp.concatenate([sums, sqs], axis=0)

    @pl.when(i == 0)
    def _():
        stats_ref[...] = new

    @pl.when(i > 0)
    def _():
        stats_ref[...] += new


def _k5_body(x_ref, h1_ref, h2_ref, wo_ref, stats_ref, gamma_ref,
             beta_ref, bo_ref, y_ref):
    inv_n = 1.0 / N
    mean = stats_ref[0:1, :] * inv_n
    ex2 = stats_ref[1:2, :] * inv_n
    var = ex2 - mean * mean
    wo = wo_ref[...]
    sc = gamma_ref[...] * lax.rsqrt(var + EPS)
    beff = bo_ref[...] + jnp.dot(beta_ref[...] - mean * sc, wo,
                                 preferred_element_type=jnp.float32)
    y = jnp.dot(x_ref[...] * sc[:, 0:D], wo[0:D, :],
                preferred_element_type=jnp.float32)
    y += jnp.dot(h1_ref[...] * sc[:, D:2 * D], wo[D:2 * D, :],
                 preferred_element_type=jnp.float32)
    y += jnp.dot(h2_ref[...] * sc[:, 2 * D:3 * D], wo[2 * D:3 * D, :],
                 preferred_element_type=jnp.float32)
    y_ref[...] = y + beff


def _rows_spec(i):
    return (i, 0)


def _all_spec(i):
    return (0, 0)


_f32 = jnp.float32


def _k1(deg2, x, w1):
    return pl.pallas_call(
        _k1_body,
        grid=(_GRID,),
        in_specs=[
            pl.BlockSpec((_BR, 2), _rows_spec),
            pl.BlockSpec((_BR, D), _rows_spec),
            pl.BlockSpec((D, D), _all_spec),
        ],
        out_specs=[
            pl.BlockSpec((_BR, D), _rows_spec),
            pl.BlockSpec((_BR, 1), _rows_spec),
        ],
        out_shape=[
            jax.ShapeDtypeStruct((N, D), _f32),
            jax.ShapeDtypeStruct((N, 1), _f32),
        ],
    )(deg2, x, w1)


def _k2(pa, pb, g, dinv, b, w):
    return pl.pallas_call(
        _k2_body,
        grid=(_GRID,),
        in_specs=[
            pl.BlockSpec((_BR, D), _rows_spec),
            pl.BlockSpec((_BR, D), _rows_spec),
            pl.BlockSpec((_BR, D), _rows_spec),
            pl.BlockSpec((_BR, 1), _rows_spec),
            pl.BlockSpec((1, D), _all_spec),
            pl.BlockSpec((D, D), _all_spec),
        ],
        out_specs=[
            pl.BlockSpec((_BR, D), _rows_spec),
            pl.BlockSpec((_BR, D), _rows_spec),
        ],
        out_shape=[
            jax.ShapeDtypeStruct((N, D), _f32),
            jax.ShapeDtypeStruct((N, D), _f32),
        ],
    )(pa, pb, g, dinv, b, w)


def _k3(pa, pb, g, dinv, b, x, h1):
    return pl.pallas_call(
        _k3_body,
        grid=(_GRID,),
        in_specs=[
            pl.BlockSpec((_BR, D), _rows_spec),
            pl.BlockSpec((_BR, D), _rows_spec),
            pl.BlockSpec((_BR, D), _rows_spec),
            pl.BlockSpec((_BR, 1), _rows_spec),
            pl.BlockSpec((1, D), _all_spec),
            pl.BlockSpec((_BR, D), _rows_spec),
            pl.BlockSpec((_BR, D), _rows_spec),
        ],
        out_specs=[
            pl.BlockSpec((_BR, D), _rows_spec),
            pl.BlockSpec((2, 3 * D), _all_spec),
        ],
        out_shape=[
            jax.ShapeDtypeStruct((N, D), _f32),
            jax.ShapeDtypeStruct((2, 3 * D), _f32),
        ],
    )(pa, pb, g, dinv, b, x, h1)


def _k5(x, h1, h2, wo, stats, gamma, beta, bo):
    return pl.pallas_call(
        _k5_body,
        grid=(_GRID,),
        in_specs=[
            pl.BlockSpec((_BR, D), _rows_spec),
            pl.BlockSpec((_BR, D), _rows_spec),
            pl.BlockSpec((_BR, D), _rows_spec),
            pl.BlockSpec((3 * D, D), _all_spec),
            pl.BlockSpec((2, 3 * D), _all_spec),
            pl.BlockSpec((1, 3 * D), _all_spec),
            pl.BlockSpec((1, 3 * D), _all_spec),
            pl.BlockSpec((1, D), _all_spec),
        ],
        out_specs=pl.BlockSpec((_BR, D), _rows_spec),
        out_shape=jax.ShapeDtypeStruct((N, D), _f32),
    )(x, h1, h2, wo, stats, gamma, beta, bo)


def kernel(x, edge_index, W1, b1, W2, b2, bn_gamma, bn_beta, Wo, bo):
    src = edge_index[0]
    dst = edge_index[1]
    src4 = src.reshape(NW, NCHUNK, CH, KE)
    dst4 = dst.reshape(NW, NCHUNK, CH, KE)
    dst3d = dst.reshape(NW, ITERS_D, KD)
    ones_kd = jnp.ones((KD,), _f32)
    zeros_ds = jnp.zeros((DSLICE,), _f32)
    zrows = jnp.zeros((ZR, D), _f32)

    degflat = _deg_sc(dst3d, ones_kd, zeros_ds)
    deg2 = degflat.reshape(2, NP)[:, :N].T

    g1, dinv = _k1(deg2, x, W1)
    p1 = _scatter_sc(g1, src4, dst4, zrows)
    h1, g2 = _k2(p1[:N], p1[N:], g1, dinv, b1.reshape(1, D), W2)
    p2 = _scatter_sc(g2, src4, dst4, zrows)
    h2, stats = _k3(p2[:N], p2[N:], g2, dinv, b2.reshape(1, D),
                    x, h1)
    return _k5(x, h1, h2, Wo, stats, bn_gamma.reshape(1, 3 * D),
               bn_beta.reshape(1, 3 * D), bo.reshape(1, D))


# speedup vs baseline: 1.0278x; 1.0169x over previous
import functools

import jax
import jax.numpy as jnp
from jax import lax
from jax.experimental import pallas as pl
from jax.experimental.pallas import tpu as pltpu
from jax.experimental.pallas import tpu_sc as plsc

N = 10000
E = 320000
D = 128
EPS = 1e-5

NC = 2
NS = 16
NW = NC * NS

EPW = E // NW
KE = 100
CH = 25
NCHUNK = EPW // (KE * CH)
KD = 125
ITERS_D = EPW // KD
NP = 10240
DSLICE = NP // NS
NT10 = 10
RPT = N // NT10
ZR = 200

_mesh = plsc.VectorSubcoreMesh(core_axis_name="c", subcore_axis_name="s")


@functools.partial(
    pl.kernel,
    out_type=jax.ShapeDtypeStruct((2 * NP,), jnp.float32),
    mesh=_mesh,
    scratch_types=[
        pltpu.VMEM((ITERS_D, KD), jnp.int32),
        pltpu.VMEM((KD,), jnp.float32),
        pltpu.VMEM_SHARED((NP,), jnp.float32),
        pltpu.SemaphoreType.DMA,
    ],
)
def _deg_sc(dst3_hbm, ones_hbm, zeros_hbm, out_hbm, idx_v, ones_v, deg_sh,
            sem):
    c = lax.axis_index("c")
    s = lax.axis_index("s")
    wid = s * NC + c
    pltpu.sync_copy(dst3_hbm.at[wid], idx_v)
    pltpu.sync_copy(ones_hbm, ones_v)
    pltpu.sync_copy(zeros_hbm, deg_sh.at[pl.ds(s * DSLICE, DSLICE)])
    plsc.subcore_barrier()

    def body(j, carry):
        for b in range(4):
            pltpu.async_copy(ones_v, deg_sh.at[idx_v.at[4 * j + b]], sem,
                             add=True)
        for b in range(4):
            pltpu.make_async_copy(ones_v, deg_sh.at[idx_v.at[0]],
                                  sem).wait()
        return carry

    lax.fori_loop(0, ITERS_D // 4, body, 0)
    plsc.subcore_barrier()
    pltpu.sync_copy(deg_sh.at[pl.ds(s * DSLICE, DSLICE)],
                    out_hbm.at[pl.ds(c * NP + s * DSLICE, DSLICE)])


@functools.partial(
    pl.kernel,
    out_type=jax.ShapeDtypeStruct((2 * N, D), jnp.float32),
    mesh=_mesh,
    scratch_types=[
        pltpu.VMEM((CH, KE), jnp.int32),
        pltpu.VMEM((CH, KE), jnp.int32),
        [pltpu.VMEM((KE, D), jnp.float32)] * 3,
        pltpu.VMEM_SHARED((N, D), jnp.float32),
        [pltpu.SemaphoreType.DMA] * 3,
    ],
)
def _scatter_sc(g_hbm, src4_hbm, dst4_hbm, zrows_hbm, out_hbm,
                idx_s, idx_d, rows, acc_sh, semg):
    c = lax.axis_index("c")
    s = lax.axis_index("s")
    wid = s * NC + c

    pltpu.sync_copy(src4_hbm.at[wid, 0], idx_s)
    pltpu.sync_copy(dst4_hbm.at[wid, 0], idx_d)
    for b in range(3):
        pltpu.async_copy(g_hbm.at[idx_s.at[b]], rows[b], semg[b])

    @pl.when(s < NT10)
    def _():
        for j in range(RPT // ZR):
            pltpu.sync_copy(zrows_hbm,
                            acc_sh.at[pl.ds(s * RPT + j * ZR, ZR)])

    plsc.subcore_barrier()

    for ck in range(NCHUNK):
        if ck > 0:
            pltpu.sync_copy(src4_hbm.at[wid, ck], idx_s)
            pltpu.sync_copy(dst4_hbm.at[wid, ck], idx_d)
            for b in range(3):
                pltpu.async_copy(g_hbm.at[idx_s.at[b]], rows[b], semg[b])

        def body(j, carry):
            for b in range(3):
                i = 3 * j + b
                pltpu.make_async_copy(g_hbm.at[idx_s.at[0]], rows[b],
                                      semg[b]).wait()
                pltpu.sync_copy(rows[b], acc_sh.at[idx_d.at[i]],
                                add=True)
                pltpu.async_copy(g_hbm.at[idx_s.at[i + 3]], rows[b],
                                 semg[b])
            return carry

        lax.fori_loop(0, CH // 3 - 1, body, 0)
        for i in range(CH - 4, CH):
            b = i % 3
            pltpu.make_async_copy(g_hbm.at[idx_s.at[0]], rows[b],
                                  semg[b]).wait()
            pltpu.sync_copy(rows[b], acc_sh.at[idx_d.at[i]], add=True)
            if i == CH - 4:
                pltpu.async_copy(g_hbm.at[idx_s.at[CH - 1]], rows[b],
                                 semg[b])

    plsc.subcore_barrier()

    @pl.when(s < NT10)
    def _():
        for j in range(RPT // ZR):
            r0 = s * RPT + j * ZR
            pltpu.sync_copy(acc_sh.at[pl.ds(r0, ZR)],
                            out_hbm.at[pl.ds(c * N + r0, ZR)])


_BR = 1000
_GRID = N // _BR


def _k1_body(deg_ref, x_ref, w1_ref, g1_ref, dinv_ref):
    deg = deg_ref[:, 0] + deg_ref[:, 1] + 1.0
    dinv = lax.rsqrt(jnp.maximum(deg, 1.0))
    dinv_ref[...] = dinv[:, None]
    h = jnp.dot(x_ref[...], w1_ref[...], preferred_element_type=jnp.float32)
    g1_ref[...] = h * dinv[:, None]


def _k2_body(pa_ref, pb_ref, g_ref, dinv_ref, b_ref, w_ref, h_ref, gn_ref):
    dinv = dinv_ref[...]
    h = jax.nn.relu(dinv * (pa_ref[...] + pb_ref[...] + g_ref[...])
                    + b_ref[...])
    h_ref[...] = h
    gn_ref[...] = jnp.dot(h, w_ref[...],
                          preferred_element_type=jnp.float32) * dinv


def _k3_body(pa_ref, pb_ref, g_ref, dinv_ref, b_ref, x_ref, h1_ref,
             h2_ref, stats_ref):
    i = pl.program_id(0)
    dinv = dinv_ref[...]
    h2 = jax.nn.relu(dinv * (pa_ref[...] + pb_ref[...] + g_ref[...])
                     + b_ref[...])
    h2_ref[...] = h2
    x = x_ref[...]
    h1 = h1_ref[...]
    sums = jnp.concatenate([x.sum(0), h1.sum(0), h2.sum(0)])[None, :]
    sqs = jnp.concatenate([(x * x).sum(0), (h1 * h1).sum(0),
                           (h2 * h2).sum(0)])[None, :]
    new = jnp.concatenate([sums, sqs], axis=0)

    @pl.when(i == 0)
    def _():
        stats_ref[...] = new

    @pl.when(i > 0)
    def _():
        stats_ref[...] += new


def _k5_body(x_ref, h1_ref, h2_ref, wo_ref, stats_ref, gamma_ref,
             beta_ref, bo_ref, y_ref):
    inv_n = 1.0 / N
    mean = stats_ref[0:1, :] * inv_n
    ex2 = stats_ref[1:2, :] * inv_n
    var = ex2 - mean * mean
    wo = wo_ref[...]
    sc = gamma_ref[...] * lax.rsqrt(var + EPS)
    beff = bo_ref[...] + jnp.dot(beta_ref[...] - mean * sc, wo,
                                 preferred_element_type=jnp.float32)
    y = jnp.dot(x_ref[...] * sc[:, 0:D], wo[0:D, :],
                preferred_element_type=jnp.float32)
    y += jnp.dot(h1_ref[...] * sc[:, D:2 * D], wo[D:2 * D, :],
                 preferred_element_type=jnp.float32)
    y += jnp.dot(h2_ref[...] * sc[:, 2 * D:3 * D], wo[2 * D:3 * D, :],
                 preferred_element_type=jnp.float32)
    y_ref[...] = y + beff


def _rows_spec(i):
    return (i, 0)


def _all_spec(i):
    return (0, 0)


_f32 = jnp.float32


def _k1(deg2, x, w1):
    return pl.pallas_call(
        _k1_body,
        grid=(_GRID,),
        in_specs=[
            pl.BlockSpec((_BR, 2), _rows_spec),
            pl.BlockSpec((_BR, D), _rows_spec),
            pl.BlockSpec((D, D), _all_spec),
        ],
        out_specs=[
            pl.BlockSpec((_BR, D), _rows_spec),
            pl.BlockSpec((_BR, 1), _rows_spec),
        ],
        out_shape=[
            jax.ShapeDtypeStruct((N, D), _f32),
            jax.ShapeDtypeStruct((N, 1), _f32),
        ],
    )(deg2, x, w1)


def _k2(pa, pb, g, dinv, b, w):
    return pl.pallas_call(
        _k2_body,
        grid=(_GRID,),
        in_specs=[
            pl.BlockSpec((_BR, D), _rows_spec),
            pl.BlockSpec((_BR, D), _rows_spec),
            pl.BlockSpec((_BR, D), _rows_spec),
            pl.BlockSpec((_BR, 1), _rows_spec),
            pl.BlockSpec((1, D), _all_spec),
            pl.BlockSpec((D, D), _all_spec),
        ],
        out_specs=[
            pl.BlockSpec((_BR, D), _rows_spec),
            pl.BlockSpec((_BR, D), _rows_spec),
        ],
        out_shape=[
            jax.ShapeDtypeStruct((N, D), _f32),
            jax.ShapeDtypeStruct((N, D), _f32),
        ],
    )(pa, pb, g, dinv, b, w)


def _k3(pa, pb, g, dinv, b, x, h1):
    return pl.pallas_call(
        _k3_body,
        grid=(_GRID,),
        in_specs=[
            pl.BlockSpec((_BR, D), _rows_spec),
            pl.BlockSpec((_BR, D), _rows_spec),
            pl.BlockSpec((_BR, D), _rows_spec),
            pl.BlockSpec((_BR, 1), _rows_spec),
            pl.BlockSpec((1, D), _all_spec),
            pl.BlockSpec((_BR, D), _rows_spec),
            pl.BlockSpec((_BR, D), _rows_spec),
        ],
        out_specs=[
            pl.BlockSpec((_BR, D), _rows_spec),
            pl.BlockSpec((2, 3 * D), _all_spec),
        ],
        out_shape=[
            jax.ShapeDtypeStruct((N, D), _f32),
            jax.ShapeDtypeStruct((2, 3 * D), _f32),
        ],
    )(pa, pb, g, dinv, b, x, h1)


def _k5(x, h1, h2, wo, stats, gamma, beta, bo):
    return pl.pallas_call(
        _k5_body,
        grid=(_GRID,),
        in_specs=[
            pl.BlockSpec((_BR, D), _rows_spec),
            pl.BlockSpec((_BR, D), _rows_spec),
            pl.BlockSpec((_BR, D), _rows_spec),
            pl.BlockSpec((3 * D, D), _all_spec),
            pl.BlockSpec((2, 3 * D), _all_spec),
            pl.BlockSpec((1, 3 * D), _all_spec),
            pl.BlockSpec((1, 3 * D), _all_spec),
            pl.BlockSpec((1, D), _all_spec),
        ],
        out_specs=pl.BlockSpec((_BR, D), _rows_spec),
        out_shape=jax.ShapeDtypeStruct((N, D), _f32),
    )(x, h1, h2, wo, stats, gamma, beta, bo)


def kernel(x, edge_index, W1, b1, W2, b2, bn_gamma, bn_beta, Wo, bo):
    src = edge_index[0]
    dst = edge_index[1]
    src4 = src.reshape(NW, NCHUNK, CH, KE)
    dst4 = dst.reshape(NW, NCHUNK, CH, KE)
    dst3d = dst.reshape(NW, ITERS_D, KD)
    ones_kd = jnp.ones((KD,), _f32)
    zeros_ds = jnp.zeros((DSLICE,), _f32)
    zrows = jnp.zeros((ZR, D), _f32)

    degflat = _deg_sc(dst3d, ones_kd, zeros_ds)
    deg2 = degflat.reshape(2, NP)[:, :N].T

    g1, dinv = _k1(deg2, x, W1)
    p1 = _scatter_sc(g1, src4, dst4, zrows)
    h1, g2 = _k2(p1[:N], p1[N:], g1, dinv, b1.reshape(1, D), W2)
    p2 = _scatter_sc(g2, src4, dst4, zrows)
    h2, stats = _k3(p2[:N], p2[N:], g2, dinv, b2.reshape(1, D),
                    x, h1)
    return _k5(x, h1, h2, Wo, stats, bn_gamma.reshape(1, 3 * D),
               bn_beta.reshape(1, 3 * D), bo.reshape(1, D))


# speedup vs baseline: 1.0287x; 1.0009x over previous
import functools

import jax
import jax.numpy as jnp
from jax import lax
from jax.experimental import pallas as pl
from jax.experimental.pallas import tpu as pltpu
from jax.experimental.pallas import tpu_sc as plsc

N = 10000
E = 320000
D = 128
EPS = 1e-5

NC = 2
NS = 16
NW = NC * NS

EPW = E // NW
KE = 100
CH = 25
NCHUNK = EPW // (KE * CH)
KD = 125
ITERS_D = EPW // KD
NP = 10240
DSLICE = NP // NS
NT10 = 10
RPT = N // NT10
ZR = 200

_mesh = plsc.VectorSubcoreMesh(core_axis_name="c", subcore_axis_name="s",
                               num_cores=NC, num_subcores=NS)


@functools.partial(
    pl.kernel,
    out_type=jax.ShapeDtypeStruct((2 * NP,), jnp.float32),
    mesh=_mesh,
    scratch_types=[
        pltpu.VMEM((ITERS_D, KD), jnp.int32),
        pltpu.VMEM((KD,), jnp.float32),
        pltpu.VMEM_SHARED((NP,), jnp.float32),
        pltpu.SemaphoreType.DMA,
    ],
)
def _deg_sc(dst3_hbm, ones_hbm, zeros_hbm, out_hbm, idx_v, ones_v, deg_sh,
            sem):
    c = lax.axis_index("c")
    s = lax.axis_index("s")
    wid = s * NC + c
    pltpu.sync_copy(dst3_hbm.at[wid], idx_v)
    pltpu.sync_copy(ones_hbm, ones_v)
    pltpu.sync_copy(zeros_hbm, deg_sh.at[pl.ds(s * DSLICE, DSLICE)])
    plsc.subcore_barrier()

    def body(j, carry):
        for b in range(4):
            pltpu.async_copy(ones_v, deg_sh.at[idx_v.at[4 * j + b]], sem,
                             add=True)
        for b in range(4):
            pltpu.make_async_copy(ones_v, deg_sh.at[idx_v.at[0]],
                                  sem).wait()
        return carry

    lax.fori_loop(0, ITERS_D // 4, body, 0)
    plsc.subcore_barrier()
    pltpu.sync_copy(deg_sh.at[pl.ds(s * DSLICE, DSLICE)],
                    out_hbm.at[pl.ds(c * NP + s * DSLICE, DSLICE)])


@functools.partial(
    pl.kernel,
    out_type=jax.ShapeDtypeStruct((2 * N, D), jnp.float32),
    mesh=_mesh,
    scratch_types=[
        pltpu.VMEM((CH, KE), jnp.int32),
        pltpu.VMEM((CH, KE), jnp.int32),
        [pltpu.VMEM((KE, D), jnp.float32)] * 3,
        pltpu.VMEM_SHARED((N, D), jnp.float32),
        [pltpu.SemaphoreType.DMA] * 3,
    ],
)
def _scatter_sc(g_hbm, src4_hbm, dst4_hbm, zrows_hbm, out_hbm,
                idx_s, idx_d, rows, acc_sh, semg):
    c = lax.axis_index("c")
    s = lax.axis_index("s")
    wid = s * NC + c

    pltpu.sync_copy(src4_hbm.at[wid, 0], idx_s)
    pltpu.sync_copy(dst4_hbm.at[wid, 0], idx_d)
    for b in range(3):
        pltpu.async_copy(g_hbm.at[idx_s.at[b]], rows[b], semg[b])

    @pl.when(s < NT10)
    def _():
        for j in range(RPT // ZR):
            pltpu.sync_copy(zrows_hbm,
                            acc_sh.at[pl.ds(s * RPT + j * ZR, ZR)])

    plsc.subcore_barrier()

    for ck in range(NCHUNK):
        if ck > 0:
            pltpu.sync_copy(src4_hbm.at[wid, ck], idx_s)
            pltpu.sync_copy(dst4_hbm.at[wid, ck], idx_d)
            for b in range(3):
                pltpu.async_copy(g_hbm.at[idx_s.at[b]], rows[b], semg[b])

        def body(j, carry):
            for b in range(3):
                i = 3 * j + b
                pltpu.make_async_copy(g_hbm.at[idx_s.at[0]], rows[b],
                                      semg[b]).wait()
                pltpu.sync_copy(rows[b], acc_sh.at[idx_d.at[i]],
                                add=True)
                pltpu.async_copy(g_hbm.at[idx_s.at[i + 3]], rows[b],
                                 semg[b])
            return carry

        lax.fori_loop(0, CH // 3 - 1, body, 0)
        for i in range(CH - 4, CH):
            b = i % 3
            pltpu.make_async_copy(g_hbm.at[idx_s.at[0]], rows[b],
                                  semg[b]).wait()
            pltpu.sync_copy(rows[b], acc_sh.at[idx_d.at[i]], add=True)
            if i == CH - 4:
                pltpu.async_copy(g_hbm.at[idx_s.at[CH - 1]], rows[b],
                                 semg[b])

    plsc.subcore_barrier()

    @pl.when(s < NT10)
    def _():
        for j in range(RPT // ZR):
            r0 = s * RPT + j * ZR
            pltpu.sync_copy(acc_sh.at[pl.ds(r0, ZR)],
                            out_hbm.at[pl.ds(c * N + r0, ZR)])


_BR = 1000
_GRID = N // _BR


def _k1_body(deg_ref, x_ref, w1_ref, g1_ref, dinv_ref):
    deg = deg_ref[:, 0] + deg_ref[:, 1] + 1.0
    dinv = lax.rsqrt(jnp.maximum(deg, 1.0))
    dinv_ref[...] = dinv[:, None]
    h = jnp.dot(x_ref[...], w1_ref[...], preferred_element_type=jnp.float32)
    g1_ref[...] = h * dinv[:, None]


def _k2_body(pa_ref, pb_ref, g_ref, dinv_ref, b_ref, w_ref, h_ref, gn_ref):
    dinv = dinv_ref[...]
    h = jax.nn.relu(dinv * (pa_ref[...] + pb_ref[...] + g_ref[...])
                    + b_ref[...])
    h_ref[...] = h
    gn_ref[...] = jnp.dot(h, w_ref[...],
                          preferred_element_type=jnp.float32) * dinv


def _k3_body(pa_ref, pb_ref, g_ref, dinv_ref, b_ref, x_ref, h1_ref,
             h2_ref, stats_ref):
    i = pl.program_id(0)
    dinv = dinv_ref[...]
    h2 = jax.nn.relu(dinv * (pa_ref[...] + pb_ref[...] + g_ref[...])
                     + b_ref[...])
    h2_ref[...] = h2
    x = x_ref[...]
    h1 = h1_ref[...]
    sums = jnp.concatenate([x.sum(0), h1.sum(0), h2.sum(0)])[None, :]
    sqs = jnp.concatenate([(x * x).sum(0), (h1 * h1).sum(0),
                           (h2 * h2).sum(0)])[None, :]
    new = jnp.concatenate([sums, sqs], axis=0)

    @pl.when(i == 0)
    def _():
        stats_ref[...] = new

    @pl.when(i > 0)
    def _():
        stats_ref[...] += new


def _k5_body(x_ref, h1_ref, h2_ref, wo_ref, stats_ref, gamma_ref,
             beta_ref, bo_ref, y_ref):
    inv_n = 1.0 / N
    mean = stats_ref[0:1, :] * inv_n
    ex2 = stats_ref[1:2, :] * inv_n
    var = ex2 - mean * mean
    wo = wo_ref[...]
    sc = gamma_ref[...] * lax.rsqrt(var + EPS)
    beff = bo_ref[...] + jnp.dot(beta_ref[...] - mean * sc, wo,
                                 preferred_element_type=jnp.float32)
    y = jnp.dot(x_ref[...] * sc[:, 0:D], wo[0:D, :],
                preferred_element_type=jnp.float32)
    y += jnp.dot(h1_ref[...] * sc[:, D:2 * D], wo[D:2 * D, :],
                 preferred_element_type=jnp.float32)
    y += jnp.dot(h2_ref[...] * sc[:, 2 * D:3 * D], wo[2 * D:3 * D, :],
                 preferred_element_type=jnp.float32)
    y_ref[...] = y + beff


def _rows_spec(i):
    return (i, 0)


def _all_spec(i):
    return (0, 0)


_f32 = jnp.float32


def _k1(deg2, x, w1):
    return pl.pallas_call(
        _k1_body,
        grid=(_GRID,),
        in_specs=[
            pl.BlockSpec((_BR, 2), _rows_spec),
            pl.BlockSpec((_BR, D), _rows_spec),
            pl.BlockSpec((D, D), _all_spec),
        ],
        out_specs=[
            pl.BlockSpec((_BR, D), _rows_spec),
            pl.BlockSpec((_BR, 1), _rows_spec),
        ],
        out_shape=[
            jax.ShapeDtypeStruct((N, D), _f32),
            jax.ShapeDtypeStruct((N, 1), _f32),
        ],
    )(deg2, x, w1)


def _k2(pa, pb, g, dinv, b, w):
    return pl.pallas_call(
        _k2_body,
        grid=(_GRID,),
        in_specs=[
            pl.BlockSpec((_BR, D), _rows_spec),
            pl.BlockSpec((_BR, D), _rows_spec),
            pl.BlockSpec((_BR, D), _rows_spec),
            pl.BlockSpec((_BR, 1), _rows_spec),
            pl.BlockSpec((1, D), _all_spec),
            pl.BlockSpec((D, D), _all_spec),
        ],
        out_specs=[
            pl.BlockSpec((_BR, D), _rows_spec),
            pl.BlockSpec((_BR, D), _rows_spec),
        ],
        out_shape=[
            jax.ShapeDtypeStruct((N, D), _f32),
            jax.ShapeDtypeStruct((N, D), _f32),
        ],
    )(pa, pb, g, dinv, b, w)


def _k3(pa, pb, g, dinv, b, x, h1):
    return pl.pallas_call(
        _k3_body,
        grid=(_GRID,),
        in_specs=[
            pl.BlockSpec((_BR, D), _rows_spec),
            pl.BlockSpec((_BR, D), _rows_spec),
            pl.BlockSpec((_BR, D), _rows_spec),
            pl.BlockSpec((_BR, 1), _rows_spec),
            pl.BlockSpec((1, D), _all_spec),
            pl.BlockSpec((_BR, D), _rows_spec),
            pl.BlockSpec((_BR, D), _rows_spec),
        ],
        out_specs=[
            pl.BlockSpec((_BR, D), _rows_spec),
            pl.BlockSpec((2, 3 * D), _all_spec),
        ],
        out_shape=[
            jax.ShapeDtypeStruct((N, D), _f32),
            jax.ShapeDtypeStruct((2, 3 * D), _f32),
        ],
    )(pa, pb, g, dinv, b, x, h1)


def _k5(x, h1, h2, wo, stats, gamma, beta, bo):
    return pl.pallas_call(
        _k5_body,
        grid=(_GRID,),
        in_specs=[
            pl.BlockSpec((_BR, D), _rows_spec),
            pl.BlockSpec((_BR, D), _rows_spec),
            pl.BlockSpec((_BR, D), _rows_spec),
            pl.BlockSpec((3 * D, D), _all_spec),
            pl.BlockSpec((2, 3 * D), _all_spec),
            pl.BlockSpec((1, 3 * D), _all_spec),
            pl.BlockSpec((1, 3 * D), _all_spec),
            pl.BlockSpec((1, D), _all_spec),
        ],
        out_specs=pl.BlockSpec((_BR, D), _rows_spec),
        out_shape=jax.ShapeDtypeStruct((N, D), _f32),
    )(x, h1, h2, wo, stats, gamma, beta, bo)


def kernel(x, edge_index, W1, b1, W2, b2, bn_gamma, bn_beta, Wo, bo):
    src = edge_index[0]
    dst = edge_index[1]
    src4 = src.reshape(NW, NCHUNK, CH, KE)
    dst4 = dst.reshape(NW, NCHUNK, CH, KE)
    dst3d = dst.reshape(NW, ITERS_D, KD)
    ones_kd = jnp.ones((KD,), _f32)
    zeros_ds = jnp.zeros((DSLICE,), _f32)
    zrows = jnp.zeros((ZR, D), _f32)

    degflat = _deg_sc(dst3d, ones_kd, zeros_ds)
    deg2 = degflat.reshape(2, NP)[:, :N].T

    g1, dinv = _k1(deg2, x, W1)
    p1 = _scatter_sc(g1, src4, dst4, zrows)
    h1, g2 = _k2(p1[:N], p1[N:], g1, dinv, b1.reshape(1, D), W2)
    p2 = _scatter_sc(g2, src4, dst4, zrows)
    h2, stats = _k3(p2[:N], p2[N:], g2, dinv, b2.reshape(1, D),
                    x, h1)
    return _k5(x, h1, h2, Wo, stats, bn_gamma.reshape(1, 3 * D),
               bn_beta.reshape(1, 3 * D), bo.reshape(1, D))
